# baseline (speedup 1.0000x reference)
import functools

import jax
import jax.numpy as jnp
from jax import lax
from jax.experimental import pallas as pl
from jax.experimental.pallas import tpu as pltpu
from jax.experimental.pallas import tpu_sc as plsc

GAMMA = 12.0
B = 16384
D = 64
L = 16
NC = 2
NS = 16
NW = NC * NS
BPW = B // NW
CHUNK = 128
NCHUNK = BPW // CHUNK

_mesh = plsc.VectorSubcoreMesh(core_axis_name="c", subcore_axis_name="s")
_params = pltpu.CompilerParams(
    needs_layout_passes=False, use_tc_tiling_on_sc=False)


@functools.partial(
    pl.kernel,
    mesh=_mesh,
    compiler_params=_params,
    out_type=jax.ShapeDtypeStruct((2 * B, D), jnp.float32),
    scratch_types=[
        pltpu.VMEM((3 * BPW,), jnp.int32),
        pltpu.VMEM((NCHUNK, CHUNK), jnp.int32),
        pltpu.VMEM((NCHUNK, CHUNK), jnp.int32),
        pltpu.VMEM((2, CHUNK, D), jnp.float32),
        pltpu.VMEM((2, CHUNK, D), jnp.float32),
        pltpu.SemaphoreType.DMA,
        pltpu.SemaphoreType.DMA,
    ],
)
def _ht_gather(sample_flat, ent, ht, samp_v, hidx, tidx, hrows, trows,
               sem0, sem1):
    wid = lax.axis_index("s") * NC + lax.axis_index("c")
    base = wid * BPW

    pltpu.sync_copy(sample_flat.at[pl.ds(base * 3, 3 * BPW)], samp_v)

    lane = lax.iota(jnp.int32, 16)
    sems = (sem0, sem1)

    def unzip(g, _):
        flat = (g * L + lane) * 3
        c = g // (CHUNK // L)
        o = (g % (CHUNK // L)) * L
        hidx[c, pl.ds(o, L)] = plsc.load_gather(samp_v, [flat])
        tidx[c, pl.ds(o, L)] = plsc.load_gather(samp_v, [flat + 2])
        return 0

    lax.fori_loop(0, BPW // L, unzip, 0)

    def fire(c, b):
        pltpu.async_copy(ent.at[hidx.at[c]], hrows.at[b], sems[b])
        pltpu.async_copy(ent.at[tidx.at[c]], trows.at[b], sems[b])

    def drain(b):
        dummy = ent.at[pl.ds(0, CHUNK)]
        pltpu.make_async_copy(dummy, hrows.at[b], sems[b]).wait()
        pltpu.make_async_copy(dummy, trows.at[b], sems[b]).wait()

    fire(0, 0)
    for c in range(NCHUNK):
        if c + 1 < NCHUNK:
            fire(c + 1, (c + 1) % 2)
        drain(c % 2)
        row0 = base + c * CHUNK
        pltpu.sync_copy(hrows.at[c % 2], ht.at[pl.ds(row0, CHUNK)])
        pltpu.sync_copy(trows.at[c % 2], ht.at[pl.ds(B + row0, CHUNK)])


@functools.partial(
    pl.kernel,
    mesh=_mesh,
    compiler_params=_params,
    out_type=jax.ShapeDtypeStruct((B,), jnp.float32),
    scratch_types=[
        pltpu.VMEM((3 * BPW,), jnp.int32),
        pltpu.VMEM((NCHUNK, CHUNK), jnp.int32),
        pltpu.VMEM((2, CHUNK, D), jnp.float32),
        pltpu.VMEM((2, CHUNK, D), jnp.float32),
        pltpu.VMEM((2, CHUNK, D), jnp.float32),
        pltpu.VMEM((BPW,), jnp.float32),
        pltpu.SemaphoreType.DMA,
        pltpu.SemaphoreType.DMA,
    ],
)
def _score(sample_flat, rel, ht, out, samp_v, ridx, rrows, hrows, trows,
           outv, sem0, sem1):
    wid = lax.axis_index("s") * NC + lax.axis_index("c")
    base = wid * BPW

    pltpu.sync_copy(sample_flat.at[pl.ds(base * 3, 3 * BPW)], samp_v)

    lane = lax.iota(jnp.int32, 16)
    sems = (sem0, sem1)

    def unzip(g, _):
        flat = (g * L + lane) * 3
        c = g // (CHUNK // L)
        o = (g % (CHUNK // L)) * L
        ridx[c, pl.ds(o, L)] = plsc.load_gather(samp_v, [flat + 1])
        return 0

    lax.fori_loop(0, BPW // L, unzip, 0)

    def fire(c, b):
        row0 = base + c * CHUNK
        pltpu.async_copy(rel.at[ridx.at[c]], rrows.at[b], sems[b])
        pltpu.async_copy(ht.at[pl.ds(row0, CHUNK)], hrows.at[b], sems[b])
        pltpu.async_copy(ht.at[pl.ds(B + row0, CHUNK)], trows.at[b], sems[b])

    def drain(b):
        dummy = rel.at[pl.ds(0, CHUNK)]
        pltpu.make_async_copy(dummy, rrows.at[b], sems[b]).wait()
        pltpu.make_async_copy(dummy, hrows.at[b], sems[b]).wait()
        pltpu.make_async_copy(dummy, trows.at[b], sems[b]).wait()

    def compute(c, b):
        def score_group(g, _):
            scores = jnp.zeros((L,), jnp.float32)
            for k in range(L):
                i = g * L + k
                acc = jnp.zeros((L,), jnp.float32)
                for j in range(D // L):
                    h = hrows[b, i, pl.ds(j * L, L)]
                    r = rrows[b, i, pl.ds(j * L, L)]
                    t = trows[b, i, pl.ds(j * L, L)]
                    acc = acc + jnp.abs(h + r - t)
                total = jnp.sum(acc)
                scores = jnp.where(lane == k, GAMMA - total, scores)
            outv[pl.ds(c * CHUNK + g * L, L)] = scores
            return 0

        lax.fori_loop(0, CHUNK // L, score_group, 0)

    fire(0, 0)
    for c in range(NCHUNK):
        if c + 1 < NCHUNK:
            fire(c + 1, (c + 1) % 2)
        drain(c % 2)
        compute(c, c % 2)

    pltpu.sync_copy(outv, out.at[pl.ds(base, BPW)])


def kernel(sample, entity_embedding, relation_embedding):
    flat = sample.reshape(-1)
    ht = _ht_gather(flat, entity_embedding)
    scores = _score(flat, relation_embedding, ht)
    return scores.reshape(B, 1)
